# Initial kernel scaffold; baseline (speedup 1.0000x reference)
import functools

import jax
import jax.numpy as jnp
from jax import lax
from jax.experimental import pallas as pl
from jax.experimental.pallas import tpu as pltpu
from jax.experimental.pallas import tpu_sc as plsc

_L = 16
_CHUNK = 128
_D = 128
_W = 144
_NC = 2
_NS = 16
_NW = _NC * _NS


def _make_sc_scatter(e_total, nseg):
  nchunks = e_total // _CHUNK
  outer_iters = (nchunks + _NW - 1) // _NW
  rows_per_tile = nseg // _NS
  mesh = plsc.VectorSubcoreMesh(
      core_axis_name="c", subcore_axis_name="s",
      num_cores=_NC, num_subcores=_NS)

  @functools.partial(
      pl.kernel,
      out_type=jax.ShapeDtypeStruct((_NC, nseg, _W), jnp.float32),
      mesh=mesh,
      scratch_types=[
          pltpu.VMEM((_CHUNK,), jnp.int32),
          pltpu.VMEM((_CHUNK,), jnp.int32),
          pltpu.VMEM((_CHUNK,), jnp.float32),
          pltpu.VMEM((_CHUNK, _D), jnp.float32),
          pltpu.VMEM((_CHUNK, _W), jnp.float32),
          pltpu.VMEM_SHARED((nseg, _W), jnp.float32),
          pltpu.SemaphoreType.DMA,
      ],
  )
  def k(table_hbm, src_hbm, dst_hbm, norm_hbm, out_hbm,
        src_v, dst_v, norm_v, rows_v, msg_v, acc, sem):
    cid = lax.axis_index("c")
    sid = lax.axis_index("s")
    wid = sid * _NC + cid

    zero = jnp.zeros((_L,), jnp.float32)

    def zero_body(i, carry):
      for j in range(_W // _L):
        msg_v[i, pl.ds(j * _L, _L)] = zero
      return carry

    lax.fori_loop(0, _CHUNK, zero_body, 0)

    base_row = sid * rows_per_tile
    off = 0
    while off < rows_per_tile:
      n = min(_CHUNK, rows_per_tile - off)
      pltpu.sync_copy(msg_v.at[pl.ds(0, n), :],
                      acc.at[pl.ds(base_row + off, n), :])
      off += n

    cvec = (lax.iota(jnp.int32, (_L,)) == 0).astype(jnp.float32)

    def cnt_body(i, carry):
      msg_v[i, pl.ds(_D, _L)] = cvec
      return carry

    lax.fori_loop(0, _CHUNK, cnt_body, 0)
    plsc.subcore_barrier()

    def outer(kk, carry):
      c = wid + kk * _NW

      @pl.when(c < nchunks)
      def _():
        base = pl.multiple_of(c * _CHUNK, _CHUNK)
        pltpu.sync_copy(src_hbm.at[pl.ds(base, _CHUNK)], src_v)
        pltpu.sync_copy(dst_hbm.at[pl.ds(base, _CHUNK)], dst_v)
        pltpu.sync_copy(norm_hbm.at[pl.ds(base, _CHUNK)], norm_v)
        pltpu.async_copy(table_hbm.at[src_v], rows_v, sem).wait()

        def scale(i, carry):
          nv = jnp.full((_L,), norm_v[i], jnp.float32)
          for j in range(_D // _L):
            msg_v[i, pl.ds(j * _L, _L)] = nv * rows_v[i, pl.ds(j * _L, _L)]
          return carry

        lax.fori_loop(0, _CHUNK, scale, 0)
        pltpu.sync_copy(msg_v, acc.at[dst_v], add=True)

      return carry

    lax.fori_loop(0, outer_iters, outer, 0)
    plsc.subcore_barrier()

    off = 0
    while off < rows_per_tile:
      n = min(_CHUNK, rows_per_tile - off)
      pltpu.sync_copy(acc.at[pl.ds(base_row + off, n), :],
                      msg_v.at[pl.ds(0, n), :])
      pltpu.sync_copy(msg_v.at[pl.ds(0, n), :],
                      out_hbm.at[cid, pl.ds(base_row + off, n), :])
      off += n

  return k


def _ln(x, g, b, eps=1e-5):
  m = jnp.mean(x, axis=-1, keepdims=True)
  v = jnp.mean((x - m) ** 2, axis=-1, keepdims=True)
  return g * (x - m) / jnp.sqrt(v + eps) + b


def _center_scale(x):
  x = x - jnp.mean(x, axis=0, keepdims=True)
  return x / jnp.sqrt(1e-5 + jnp.mean(jnp.sum(x * x, axis=-1)))


def _dense(x, g, b, w, bias):
  return jnp.maximum(
      jnp.dot(_ln(x, g, b), w, preferred_element_type=jnp.float32) + bias, 0.0)


def _stage_a_body(x_ref, g_ref, b_ref, w_ref, bias_ref, ei1_ref,
                  h_ref, row1_ref):
  h_ref[...] = _dense(x_ref[...], g_ref[...], b_ref[...], w_ref[...],
                      bias_ref[...])
  ei = ei1_ref[...]
  row1_ref[...] = ei - jnp.min(ei)


def _agg_decode(acc, dec_g, dec_b, dec_w, dec_bias):
  a = acc[0] + acc[1]
  s = a[:, :_D]
  cnt = a[:, _D:_D + 1]
  agg = s / jnp.maximum(cnt, 1.0)
  return _center_scale(_dense(agg, dec_g, dec_b, dec_w, dec_bias))


def _stage_mid_body(acc_ref, dec_g, dec_b, dec_w, dec_bias,
                    bn_g, bn_b, enc_g, enc_b, enc_w, enc_bias,
                    h_ref, table_ref):
  h = _agg_decode(acc_ref[...], dec_g[...], dec_b[...], dec_w[...],
                  dec_bias[...])
  h_ref[...] = h
  t = jnp.maximum(bn_g[...] * h / jnp.sqrt(1.0 + 1e-5) + bn_b[...], 0.0)
  table_ref[...] = _dense(t, enc_g[...], enc_b[...], enc_w[...], enc_bias[...])


def _stage_d_body(acc_ref, dec_g, dec_b, dec_w, dec_bias, h1_ref,
                  w1_ref, b1_ref, lng_ref, lnb_ref, w2_ref, b2_ref,
                  score_ref, h3_ref):
  h3 = _agg_decode(acc_ref[...], dec_g[...], dec_b[...], dec_w[...],
                   dec_bias[...])
  h3_ref[...] = h3
  xc = jnp.concatenate([h1_ref[...], h3], axis=1)
  hcl = jnp.maximum(
      jnp.dot(xc, w1_ref[...], preferred_element_type=jnp.float32)
      + b1_ref[...], 0.0)
  hcl = _ln(hcl, lng_ref[...], lnb_ref[...])
  score_ref[...] = (jnp.dot(hcl, w2_ref[...],
                            preferred_element_type=jnp.float32) + b2_ref[...])


def _r2(v):
  return v.reshape(1, -1)


def kernel(x, edge_index, norm, params):
  n, _ = x.shape
  e = edge_index.shape[1]
  nseg = 10000
  p = params

  row0 = edge_index[0]
  ei1_2d = edge_index[1].reshape(e // _D, _D)

  c0, c1, c2 = p['v2e0'], p['e2v0'], p['v2e1']

  h0, row1_2d = pl.pallas_call(
      _stage_a_body,
      out_shape=[
          jax.ShapeDtypeStruct((n, _D), jnp.float32),
          jax.ShapeDtypeStruct((e // _D, _D), jnp.int32),
      ],
  )(x, _r2(c0['enc_g']), _r2(c0['enc_b']), c0['enc_W'], _r2(c0['enc_bias']),
    ei1_2d)
  row1 = row1_2d.reshape(e)

  sc_scatter = _make_sc_scatter(e, nseg)

  acc1 = sc_scatter(h0, row0, row1, norm)

  h1, table1 = pl.pallas_call(
      _stage_mid_body,
      out_shape=[
          jax.ShapeDtypeStruct((nseg, _D), jnp.float32),
          jax.ShapeDtypeStruct((nseg, _D), jnp.float32),
      ],
  )(acc1, _r2(c0['dec_g']), _r2(c0['dec_b']), c0['dec_W'], _r2(c0['dec_bias']),
    _r2(p['bnV0_g']), _r2(p['bnV0_b']),
    _r2(c1['enc_g']), _r2(c1['enc_b']), c1['enc_W'], _r2(c1['enc_bias']))

  acc2 = sc_scatter(table1, row1, row0, norm)

  node_feat, table2 = pl.pallas_call(
      _stage_mid_body,
      out_shape=[
          jax.ShapeDtypeStruct((n, _D), jnp.float32),
          jax.ShapeDtypeStruct((n, _D), jnp.float32),
      ],
  )(acc2, _r2(c1['dec_g']), _r2(c1['dec_b']), c1['dec_W'], _r2(c1['dec_bias']),
    _r2(p['bnE0_g']), _r2(p['bnE0_b']),
    _r2(c2['enc_g']), _r2(c2['enc_b']), c2['enc_W'], _r2(c2['enc_bias']))

  acc3 = sc_scatter(table2, row0, row1, norm)

  edge_score, edge_feat = pl.pallas_call(
      _stage_d_body,
      out_shape=[
          jax.ShapeDtypeStruct((nseg, 2), jnp.float32),
          jax.ShapeDtypeStruct((nseg, _D), jnp.float32),
      ],
  )(acc3, _r2(c2['dec_g']), _r2(c2['dec_b']), c2['dec_W'], _r2(c2['dec_bias']),
    h1,
    p['cls_W1'], _r2(p['cls_b1']), _r2(p['cls_lng']), _r2(p['cls_lnb']),
    p['cls_W2'], _r2(p['cls_b2']))

  return (edge_score, edge_feat, node_feat, norm)


# speedup vs baseline: 8.0591x; 8.0591x over previous
import functools

import jax
import jax.numpy as jnp
from jax import lax
from jax.experimental import pallas as pl
from jax.experimental.pallas import tpu as pltpu
from jax.experimental.pallas import tpu_sc as plsc

_L = 16
_CHUNK = 128
_D = 128
_CW = 16
_NC = 2
_NS = 16
_NW = _NC * _NS


def _make_sc_scatter(e_total, nseg):
  nchunks = e_total // _CHUNK
  ept = nchunks // _NW
  extra = nchunks - ept * _NW
  rows_per_tile = -(-nseg // _NS)
  rows_per_tile = -(-rows_per_tile // _CHUNK) * _CHUNK
  nseg_pad = rows_per_tile * _NS
  tail = nseg - (nseg // _CHUNK) * _CHUNK
  mesh = plsc.VectorSubcoreMesh(
      core_axis_name="c", subcore_axis_name="s",
      num_cores=_NC, num_subcores=_NS)

  @functools.partial(
      pl.kernel,
      out_type=(
          jax.ShapeDtypeStruct((_NC, nseg, _D), jnp.float32),
          jax.ShapeDtypeStruct((_NC, nseg, _CW), jnp.float32),
      ),
      mesh=mesh,
      compiler_params=pltpu.CompilerParams(use_tc_tiling_on_sc=False,
                                           needs_layout_passes=False),
      scratch_types=[
          pltpu.VMEM((2, 3, _CHUNK), jnp.int32),
          pltpu.VMEM((_CHUNK, _D), jnp.float32),
          pltpu.VMEM((_CHUNK, _D), jnp.float32),
          pltpu.VMEM((2, _CHUNK), jnp.int32),
          pltpu.VMEM((_CHUNK, _CW), jnp.float32),
          pltpu.VMEM_SHARED((nseg_pad, _D), jnp.float32),
          pltpu.VMEM_SHARED((nseg_pad, _CW), jnp.float32),
          pltpu.SemaphoreType.DMA,
          pltpu.SemaphoreType.DMA,
          pltpu.SemaphoreType.DMA,
          pltpu.SemaphoreType.DMA,
          pltpu.SemaphoreType.DMA,
          pltpu.SemaphoreType.DMA,
          pltpu.SemaphoreType.DMA,
          pltpu.SemaphoreType.DMA,
      ],
  )
  def k(table_hbm, pk_hbm, out_hbm, outc_hbm,
        pk_v, msg0, msg1, dst_c, cnt_v, acc, accc,
        isem0, isem1, gsem0, gsem1, ssem0, ssem1, csem0, csem1):
    cid = lax.axis_index("c")
    sid = lax.axis_index("s")
    wid = sid * _NC + cid
    msg = (msg0, msg1)
    isem = (isem0, isem1)
    gsem = (gsem0, gsem1)
    ssem = (ssem0, ssem1)
    csem = (csem0, csem1)
    cbase = wid * ept

    def issue_idx(c, j):
      pltpu.async_copy(pk_hbm.at[cbase + c], pk_v.at[j], isem[j])

    def drain_idx(j):
      pltpu.make_async_copy(pk_hbm.at[0], pk_v.at[j], isem[j]).wait()

    def issue_gather(j):
      pltpu.async_copy(table_hbm.at[pk_v.at[j, 0]], msg[j], gsem[j])

    def drain_gather(j):
      pltpu.make_async_copy(table_hbm.at[pl.ds(0, _CHUNK), :], msg[j],
                            gsem[j]).wait()

    def issue_scatter(j):
      pltpu.async_copy(msg[j], acc.at[dst_c.at[j]], ssem[j], add=True)
      pltpu.async_copy(cnt_v, accc.at[dst_c.at[j]], csem[j], add=True)

    def drain_scatter(j):
      pltpu.make_async_copy(out_hbm.at[0, pl.ds(0, _CHUNK), :], msg[j],
                            ssem[j]).wait()
      pltpu.make_async_copy(outc_hbm.at[0, pl.ds(0, _CHUNK), :], cnt_v,
                            csem[j]).wait()

    issue_idx(0, 0)
    issue_idx(1, 1)

    zero = jnp.zeros((_L,), jnp.float32)

    def zero_body(i, carry):
      for j in range(_D // _L):
        msg0[i, pl.ds(j * _L, _L)] = zero
      cnt_v[i, pl.ds(0, _L)] = zero
      return carry

    lax.fori_loop(0, _CHUNK, zero_body, 0)

    base_row = sid * rows_per_tile
    for off in range(0, rows_per_tile, _CHUNK):
      pltpu.sync_copy(msg0.at[pl.ds(0, _CHUNK), :],
                      acc.at[pl.ds(base_row + off, _CHUNK), :])
      pltpu.sync_copy(cnt_v, accc.at[pl.ds(base_row + off, _CHUNK), :])

    ones = jnp.ones((_L,), jnp.float32)

    def ones_body(i, carry):
      cnt_v[i, pl.ds(0, _L)] = ones
      return carry

    lax.fori_loop(0, _CHUNK, ones_body, 0)
    plsc.subcore_barrier()

    drain_idx(0)
    issue_gather(0)

    def do_chunk(c, b):
      drain_gather(b)

      for j in range(_CHUNK // _L):
        dst_c[b, pl.ds(j * _L, _L)] = pk_v[b, 1, pl.ds(j * _L, _L)]

      def scale(g, carry):
        nv16 = plsc.bitcast(pk_v[b, 2, pl.ds(g * _L, _L)], jnp.float32)
        for t in range(_L):
          s = nv16[t]
          row = g * _L + t
          for j in range(_D // _L):
            msg[b][row, pl.ds(j * _L, _L)] = s * msg[b][row, pl.ds(j * _L, _L)]
        return carry

      lax.fori_loop(0, _CHUNK // _L, scale, 0)

      @pl.when(c >= 1)
      def _():
        drain_scatter(1 - b)

      @pl.when(c + 1 < ept)
      def _():
        drain_idx(1 - b)
        issue_gather(1 - b)

      @pl.when(c + 2 < ept)
      def _():
        issue_idx(c + 2, b)

      issue_scatter(b)

    def outer(g, carry):
      do_chunk(g * 2, 0)
      do_chunk(g * 2 + 1, 1)
      return carry

    lax.fori_loop(0, ept // 2, outer, 0)
    if ept % 2:
      do_chunk(ept - 1, (ept - 1) % 2)
    if ept >= 1:
      drain_scatter((ept - 1) % 2)

    if extra:
      @pl.when(wid < extra)
      def _():
        xc = ept * _NW + wid
        pltpu.sync_copy(pk_hbm.at[xc], pk_v.at[0])
        pltpu.async_copy(table_hbm.at[pk_v.at[0, 0]], msg0, gsem0).wait()
        for j in range(_CHUNK // _L):
          dst_c[0, pl.ds(j * _L, _L)] = pk_v[0, 1, pl.ds(j * _L, _L)]

        def xscale(g, carry):
          nv16 = plsc.bitcast(pk_v[0, 2, pl.ds(g * _L, _L)], jnp.float32)
          for t in range(_L):
            s = nv16[t]
            row = g * _L + t
            for j in range(_D // _L):
              msg0[row, pl.ds(j * _L, _L)] = s * msg0[row, pl.ds(j * _L, _L)]
          return carry

        lax.fori_loop(0, _CHUNK // _L, xscale, 0)
        pltpu.sync_copy(msg0, acc.at[dst_c.at[0]], add=True)
        pltpu.sync_copy(cnt_v, accc.at[dst_c.at[0]], add=True)

    plsc.subcore_barrier()

    for off in range(0, rows_per_tile, _CHUNK):
      start = base_row + off

      @pl.when(start + _CHUNK <= nseg)
      def _():
        pltpu.sync_copy(acc.at[pl.ds(start, _CHUNK), :],
                        msg0.at[pl.ds(0, _CHUNK), :])
        pltpu.sync_copy(msg0.at[pl.ds(0, _CHUNK), :],
                        out_hbm.at[cid, pl.ds(start, _CHUNK), :])
        pltpu.sync_copy(accc.at[pl.ds(start, _CHUNK), :], cnt_v)
        pltpu.sync_copy(cnt_v, outc_hbm.at[cid, pl.ds(start, _CHUNK), :])

      if tail:
        @pl.when((start < nseg) & (start + _CHUNK > nseg))
        def _():
          pltpu.sync_copy(acc.at[pl.ds(start, tail), :],
                          msg0.at[pl.ds(0, tail), :])
          pltpu.sync_copy(msg0.at[pl.ds(0, tail), :],
                          out_hbm.at[cid, pl.ds(start, tail), :])
          pltpu.sync_copy(accc.at[pl.ds(start, tail), :],
                          cnt_v.at[pl.ds(0, tail), :])
          pltpu.sync_copy(cnt_v.at[pl.ds(0, tail), :],
                          outc_hbm.at[cid, pl.ds(start, tail), :])

  return k


def _ln(x, g, b, eps=1e-5):
  m = jnp.mean(x, axis=-1, keepdims=True)
  v = jnp.mean((x - m) ** 2, axis=-1, keepdims=True)
  return g * (x - m) / jnp.sqrt(v + eps) + b


def _center_scale(x):
  x = x - jnp.mean(x, axis=0, keepdims=True)
  return x / jnp.sqrt(1e-5 + jnp.mean(jnp.sum(x * x, axis=-1)))


def _dense(x, g, b, w, bias):
  return jnp.maximum(
      jnp.dot(_ln(x, g, b), w, preferred_element_type=jnp.float32) + bias, 0.0)


def _stage_a_body(x_ref, g_ref, b_ref, w_ref, bias_ref, ei0_ref, ei1_ref,
                  nrm_ref, h_ref, pk1_ref, pk2_ref):
  h_ref[...] = _dense(x_ref[...], g_ref[...], b_ref[...], w_ref[...],
                      bias_ref[...])
  row0 = ei0_ref[...]
  ei1 = ei1_ref[...]
  row1 = ei1 - jnp.min(ei1)
  nrm_bits = jax.lax.bitcast_convert_type(nrm_ref[...], jnp.int32)
  pk1_ref[...] = jnp.stack([row0, row1, nrm_bits], axis=1)
  pk2_ref[...] = jnp.stack([row1, row0, nrm_bits], axis=1)


def _agg_decode(acc, accc, dec_g, dec_b, dec_w, dec_bias):
  s = acc[0] + acc[1]
  cnt = (accc[0] + accc[1])[:, 0:1]
  agg = s / jnp.maximum(cnt, 1.0)
  return _center_scale(_dense(agg, dec_g, dec_b, dec_w, dec_bias))


def _stage_mid_body(acc_ref, accc_ref, dec_g, dec_b, dec_w, dec_bias,
                    bn_g, bn_b, enc_g, enc_b, enc_w, enc_bias,
                    h_ref, table_ref):
  h = _agg_decode(acc_ref[...], accc_ref[...], dec_g[...], dec_b[...],
                  dec_w[...], dec_bias[...])
  h_ref[...] = h
  t = jnp.maximum(bn_g[...] * h / jnp.sqrt(1.0 + 1e-5) + bn_b[...], 0.0)
  table_ref[...] = _dense(t, enc_g[...], enc_b[...], enc_w[...], enc_bias[...])


def _stage_d_body(acc_ref, accc_ref, dec_g, dec_b, dec_w, dec_bias, h1_ref,
                  w1_ref, b1_ref, lng_ref, lnb_ref, w2_ref, b2_ref,
                  score_ref, h3_ref):
  h3 = _agg_decode(acc_ref[...], accc_ref[...], dec_g[...], dec_b[...],
                   dec_w[...], dec_bias[...])
  h3_ref[...] = h3
  xc = jnp.concatenate([h1_ref[...], h3], axis=1)
  hcl = jnp.maximum(
      jnp.dot(xc, w1_ref[...], preferred_element_type=jnp.float32)
      + b1_ref[...], 0.0)
  hcl = _ln(hcl, lng_ref[...], lnb_ref[...])
  score_ref[...] = (jnp.dot(hcl, w2_ref[...],
                            preferred_element_type=jnp.float32) + b2_ref[...])


def _r2(v):
  return v.reshape(1, -1)


def kernel(x, edge_index, norm, params):
  n, _ = x.shape
  e = edge_index.shape[1]
  nseg = 10000
  nch = e // _CHUNK
  p = params

  ei0_2d = edge_index[0].reshape(nch, _CHUNK)
  ei1_2d = edge_index[1].reshape(nch, _CHUNK)
  nrm_2d = norm.reshape(nch, _CHUNK)

  c0, c1, c2 = p['v2e0'], p['e2v0'], p['v2e1']

  h0, pk1, pk2 = pl.pallas_call(
      _stage_a_body,
      out_shape=[
          jax.ShapeDtypeStruct((n, _D), jnp.float32),
          jax.ShapeDtypeStruct((nch, 3, _CHUNK), jnp.int32),
          jax.ShapeDtypeStruct((nch, 3, _CHUNK), jnp.int32),
      ],
  )(x, _r2(c0['enc_g']), _r2(c0['enc_b']), c0['enc_W'], _r2(c0['enc_bias']),
    ei0_2d, ei1_2d, nrm_2d)

  sc_scatter = _make_sc_scatter(e, nseg)

  acc1, accc1 = sc_scatter(h0, pk1)

  h1, table1 = pl.pallas_call(
      _stage_mid_body,
      out_shape=[
          jax.ShapeDtypeStruct((nseg, _D), jnp.float32),
          jax.ShapeDtypeStruct((nseg, _D), jnp.float32),
      ],
  )(acc1, accc1,
    _r2(c0['dec_g']), _r2(c0['dec_b']), c0['dec_W'], _r2(c0['dec_bias']),
    _r2(p['bnV0_g']), _r2(p['bnV0_b']),
    _r2(c1['enc_g']), _r2(c1['enc_b']), c1['enc_W'], _r2(c1['enc_bias']))

  acc2, accc2 = sc_scatter(table1, pk2)

  node_feat, table2 = pl.pallas_call(
      _stage_mid_body,
      out_shape=[
          jax.ShapeDtypeStruct((n, _D), jnp.float32),
          jax.ShapeDtypeStruct((n, _D), jnp.float32),
      ],
  )(acc2, accc2,
    _r2(c1['dec_g']), _r2(c1['dec_b']), c1['dec_W'], _r2(c1['dec_bias']),
    _r2(p['bnE0_g']), _r2(p['bnE0_b']),
    _r2(c2['enc_g']), _r2(c2['enc_b']), c2['enc_W'], _r2(c2['enc_bias']))

  acc3, accc3 = sc_scatter(table2, pk1)

  edge_score, edge_feat = pl.pallas_call(
      _stage_d_body,
      out_shape=[
          jax.ShapeDtypeStruct((nseg, 2), jnp.float32),
          jax.ShapeDtypeStruct((nseg, _D), jnp.float32),
      ],
  )(acc3, accc3,
    _r2(c2['dec_g']), _r2(c2['dec_b']), c2['dec_W'], _r2(c2['dec_bias']),
    h1,
    p['cls_W1'], _r2(p['cls_b1']), _r2(p['cls_lng']), _r2(p['cls_lnb']),
    p['cls_W2'], _r2(p['cls_b2']))

  return (edge_score, edge_feat, node_feat, norm)


# speedup vs baseline: 8.0745x; 1.0019x over previous
import functools

import jax
import jax.numpy as jnp
from jax import lax
from jax.experimental import pallas as pl
from jax.experimental.pallas import tpu as pltpu
from jax.experimental.pallas import tpu_sc as plsc

_L = 16
_CHUNK = 128
_D = 128
_CW = 16
_NC = 2
_NS = 16
_NW = _NC * _NS


def _make_sc_scatter(e_total, nseg):
  nchunks = e_total // _CHUNK
  ept = nchunks // _NW
  extra = nchunks - ept * _NW
  rows_per_tile = -(-nseg // _NS)
  rows_per_tile = -(-rows_per_tile // _CHUNK) * _CHUNK
  nseg_pad = rows_per_tile * _NS
  tail = nseg - (nseg // _CHUNK) * _CHUNK
  mesh = plsc.VectorSubcoreMesh(
      core_axis_name="c", subcore_axis_name="s",
      num_cores=_NC, num_subcores=_NS)

  @functools.partial(
      pl.kernel,
      out_type=(
          jax.ShapeDtypeStruct((_NC, nseg, _D), jnp.float32),
          jax.ShapeDtypeStruct((_NC, nseg, _CW), jnp.float32),
      ),
      mesh=mesh,
      compiler_params=pltpu.CompilerParams(use_tc_tiling_on_sc=False,
                                           needs_layout_passes=False),
      scratch_types=[
          pltpu.VMEM((2, 3, _CHUNK), jnp.int32),
          pltpu.VMEM((_CHUNK, _D), jnp.float32),
          pltpu.VMEM((_CHUNK, _D), jnp.float32),
          pltpu.VMEM((2, _CHUNK), jnp.int32),
          pltpu.VMEM((_CHUNK, _CW), jnp.float32),
          pltpu.VMEM_SHARED((nseg_pad, _D), jnp.float32),
          pltpu.VMEM_SHARED((nseg_pad, _CW), jnp.float32),
          pltpu.SemaphoreType.DMA,
          pltpu.SemaphoreType.DMA,
          pltpu.SemaphoreType.DMA,
          pltpu.SemaphoreType.DMA,
          pltpu.SemaphoreType.DMA,
          pltpu.SemaphoreType.DMA,
          pltpu.SemaphoreType.DMA,
          pltpu.SemaphoreType.DMA,
      ],
  )
  def k(table_hbm, pk_hbm, out_hbm, outc_hbm,
        pk_v, msg0, msg1, dst_c, cnt_v, acc, accc,
        isem0, isem1, gsem0, gsem1, ssem0, ssem1, csem0, csem1):
    cid = lax.axis_index("c")
    sid = lax.axis_index("s")
    wid = sid * _NC + cid
    msg = (msg0, msg1)
    isem = (isem0, isem1)
    gsem = (gsem0, gsem1)
    ssem = (ssem0, ssem1)
    csem = (csem0, csem1)
    cbase = wid * ept

    def issue_idx(c, j):
      pltpu.async_copy(pk_hbm.at[cbase + c], pk_v.at[j], isem[j])

    def drain_idx(j):
      pltpu.make_async_copy(pk_hbm.at[0], pk_v.at[j], isem[j]).wait()

    def issue_gather(j):
      pltpu.async_copy(table_hbm.at[pk_v.at[j, 0]], msg[j], gsem[j])

    def drain_gather(j):
      pltpu.make_async_copy(table_hbm.at[pl.ds(0, _CHUNK), :], msg[j],
                            gsem[j]).wait()

    def issue_scatter(j):
      pltpu.async_copy(msg[j], acc.at[dst_c.at[j]], ssem[j], add=True)
      pltpu.async_copy(cnt_v, accc.at[dst_c.at[j]], csem[j], add=True)

    def drain_scatter(j):
      pltpu.make_async_copy(out_hbm.at[0, pl.ds(0, _CHUNK), :], msg[j],
                            ssem[j]).wait()
      pltpu.make_async_copy(outc_hbm.at[0, pl.ds(0, _CHUNK), :], cnt_v,
                            csem[j]).wait()

    issue_idx(0, 0)
    issue_idx(1, 1)

    zero = jnp.zeros((_L,), jnp.float32)

    def zero_body(i, carry):
      for j in range(_D // _L):
        msg0[i, pl.ds(j * _L, _L)] = zero
      cnt_v[i, pl.ds(0, _L)] = zero
      return carry

    lax.fori_loop(0, _CHUNK, zero_body, 0)

    base_row = sid * rows_per_tile
    for off in range(0, rows_per_tile, _CHUNK):
      pltpu.sync_copy(msg0.at[pl.ds(0, _CHUNK), :],
                      acc.at[pl.ds(base_row + off, _CHUNK), :])
      pltpu.sync_copy(cnt_v, accc.at[pl.ds(base_row + off, _CHUNK), :])

    ones = jnp.ones((_L,), jnp.float32)

    def ones_body(i, carry):
      cnt_v[i, pl.ds(0, _L)] = ones
      return carry

    lax.fori_loop(0, _CHUNK, ones_body, 0)
    plsc.subcore_barrier()

    drain_idx(0)
    issue_gather(0)

    def do_chunk(c, b):
      drain_gather(b)

      for j in range(_CHUNK // _L):
        dst_c[b, pl.ds(j * _L, _L)] = pk_v[b, 1, pl.ds(j * _L, _L)]

      def scale(g, carry):
        nv16 = plsc.bitcast(pk_v[b, 2, pl.ds(g * _L, _L)], jnp.float32)
        for t in range(_L):
          s = nv16[t]
          row = g * _L + t
          for j in range(_D // _L):
            msg[b][row, pl.ds(j * _L, _L)] = s * msg[b][row, pl.ds(j * _L, _L)]
        return carry

      lax.fori_loop(0, _CHUNK // _L, scale, 0)
      issue_scatter(b)

      @pl.when(c >= 1)
      def _():
        drain_scatter(1 - b)

      @pl.when(c + 1 < ept)
      def _():
        drain_idx(1 - b)
        issue_gather(1 - b)

      @pl.when(c + 2 < ept)
      def _():
        issue_idx(c + 2, b)

    def outer(g, carry):
      do_chunk(g * 2, 0)
      do_chunk(g * 2 + 1, 1)
      return carry

    lax.fori_loop(0, ept // 2, outer, 0)
    if ept % 2:
      do_chunk(ept - 1, (ept - 1) % 2)
    if ept >= 1:
      drain_scatter((ept - 1) % 2)

    if extra:
      @pl.when(wid < extra)
      def _():
        xc = ept * _NW + wid
        pltpu.sync_copy(pk_hbm.at[xc], pk_v.at[0])
        pltpu.async_copy(table_hbm.at[pk_v.at[0, 0]], msg0, gsem0).wait()
        for j in range(_CHUNK // _L):
          dst_c[0, pl.ds(j * _L, _L)] = pk_v[0, 1, pl.ds(j * _L, _L)]

        def xscale(g, carry):
          nv16 = plsc.bitcast(pk_v[0, 2, pl.ds(g * _L, _L)], jnp.float32)
          for t in range(_L):
            s = nv16[t]
            row = g * _L + t
            for j in range(_D // _L):
              msg0[row, pl.ds(j * _L, _L)] = s * msg0[row, pl.ds(j * _L, _L)]
          return carry

        lax.fori_loop(0, _CHUNK // _L, xscale, 0)
        pltpu.sync_copy(msg0, acc.at[dst_c.at[0]], add=True)
        pltpu.sync_copy(cnt_v, accc.at[dst_c.at[0]], add=True)

    plsc.subcore_barrier()

    for off in range(0, rows_per_tile, _CHUNK):
      start = base_row + off

      @pl.when(start + _CHUNK <= nseg)
      def _():
        pltpu.sync_copy(acc.at[pl.ds(start, _CHUNK), :],
                        msg0.at[pl.ds(0, _CHUNK), :])
        pltpu.sync_copy(msg0.at[pl.ds(0, _CHUNK), :],
                        out_hbm.at[cid, pl.ds(start, _CHUNK), :])
        pltpu.sync_copy(accc.at[pl.ds(start, _CHUNK), :], cnt_v)
        pltpu.sync_copy(cnt_v, outc_hbm.at[cid, pl.ds(start, _CHUNK), :])

      if tail:
        @pl.when((start < nseg) & (start + _CHUNK > nseg))
        def _():
          pltpu.sync_copy(acc.at[pl.ds(start, tail), :],
                          msg0.at[pl.ds(0, tail), :])
          pltpu.sync_copy(msg0.at[pl.ds(0, tail), :],
                          out_hbm.at[cid, pl.ds(start, tail), :])
          pltpu.sync_copy(accc.at[pl.ds(start, tail), :],
                          cnt_v.at[pl.ds(0, tail), :])
          pltpu.sync_copy(cnt_v.at[pl.ds(0, tail), :],
                          outc_hbm.at[cid, pl.ds(start, tail), :])

  return k


def _ln(x, g, b, eps=1e-5):
  m = jnp.mean(x, axis=-1, keepdims=True)
  v = jnp.mean((x - m) ** 2, axis=-1, keepdims=True)
  return g * (x - m) / jnp.sqrt(v + eps) + b


def _center_scale(x):
  x = x - jnp.mean(x, axis=0, keepdims=True)
  return x / jnp.sqrt(1e-5 + jnp.mean(jnp.sum(x * x, axis=-1)))


def _dense(x, g, b, w, bias):
  return jnp.maximum(
      jnp.dot(_ln(x, g, b), w, preferred_element_type=jnp.float32) + bias, 0.0)


def _stage_a_body(x_ref, g_ref, b_ref, w_ref, bias_ref, ei0_ref, ei1_ref,
                  nrm_ref, h_ref, pk1_ref, pk2_ref):
  h_ref[...] = _dense(x_ref[...], g_ref[...], b_ref[...], w_ref[...],
                      bias_ref[...])
  row0 = ei0_ref[...]
  ei1 = ei1_ref[...]
  row1 = ei1 - jnp.min(ei1)
  nrm_bits = jax.lax.bitcast_convert_type(nrm_ref[...], jnp.int32)
  pk1_ref[...] = jnp.stack([row0, row1, nrm_bits], axis=1)
  pk2_ref[...] = jnp.stack([row1, row0, nrm_bits], axis=1)


def _agg_decode(acc, accc, dec_g, dec_b, dec_w, dec_bias):
  s = acc[0] + acc[1]
  cnt = (accc[0] + accc[1])[:, 0:1]
  agg = s / jnp.maximum(cnt, 1.0)
  return _center_scale(_dense(agg, dec_g, dec_b, dec_w, dec_bias))


def _stage_mid_body(acc_ref, accc_ref, dec_g, dec_b, dec_w, dec_bias,
                    bn_g, bn_b, enc_g, enc_b, enc_w, enc_bias,
                    h_ref, table_ref):
  h = _agg_decode(acc_ref[...], accc_ref[...], dec_g[...], dec_b[...],
                  dec_w[...], dec_bias[...])
  h_ref[...] = h
  t = jnp.maximum(bn_g[...] * h / jnp.sqrt(1.0 + 1e-5) + bn_b[...], 0.0)
  table_ref[...] = _dense(t, enc_g[...], enc_b[...], enc_w[...], enc_bias[...])


def _stage_d_body(acc_ref, accc_ref, dec_g, dec_b, dec_w, dec_bias, h1_ref,
                  w1_ref, b1_ref, lng_ref, lnb_ref, w2_ref, b2_ref,
                  score_ref, h3_ref):
  h3 = _agg_decode(acc_ref[...], accc_ref[...], dec_g[...], dec_b[...],
                   dec_w[...], dec_bias[...])
  h3_ref[...] = h3
  xc = jnp.concatenate([h1_ref[...], h3], axis=1)
  hcl = jnp.maximum(
      jnp.dot(xc, w1_ref[...], preferred_element_type=jnp.float32)
      + b1_ref[...], 0.0)
  hcl = _ln(hcl, lng_ref[...], lnb_ref[...])
  score_ref[...] = (jnp.dot(hcl, w2_ref[...],
                            preferred_element_type=jnp.float32) + b2_ref[...])


def _r2(v):
  return v.reshape(1, -1)


def kernel(x, edge_index, norm, params):
  n, _ = x.shape
  e = edge_index.shape[1]
  nseg = 10000
  nch = e // _CHUNK
  p = params

  ei0_2d = edge_index[0].reshape(nch, _CHUNK)
  ei1_2d = edge_index[1].reshape(nch, _CHUNK)
  nrm_2d = norm.reshape(nch, _CHUNK)

  c0, c1, c2 = p['v2e0'], p['e2v0'], p['v2e1']

  h0, pk1, pk2 = pl.pallas_call(
      _stage_a_body,
      out_shape=[
          jax.ShapeDtypeStruct((n, _D), jnp.float32),
          jax.ShapeDtypeStruct((nch, 3, _CHUNK), jnp.int32),
          jax.ShapeDtypeStruct((nch, 3, _CHUNK), jnp.int32),
      ],
  )(x, _r2(c0['enc_g']), _r2(c0['enc_b']), c0['enc_W'], _r2(c0['enc_bias']),
    ei0_2d, ei1_2d, nrm_2d)

  sc_scatter = _make_sc_scatter(e, nseg)

  acc1, accc1 = sc_scatter(h0, pk1)

  h1, table1 = pl.pallas_call(
      _stage_mid_body,
      out_shape=[
          jax.ShapeDtypeStruct((nseg, _D), jnp.float32),
          jax.ShapeDtypeStruct((nseg, _D), jnp.float32),
      ],
  )(acc1, accc1,
    _r2(c0['dec_g']), _r2(c0['dec_b']), c0['dec_W'], _r2(c0['dec_bias']),
    _r2(p['bnV0_g']), _r2(p['bnV0_b']),
    _r2(c1['enc_g']), _r2(c1['enc_b']), c1['enc_W'], _r2(c1['enc_bias']))

  acc2, accc2 = sc_scatter(table1, pk2)

  node_feat, table2 = pl.pallas_call(
      _stage_mid_body,
      out_shape=[
          jax.ShapeDtypeStruct((n, _D), jnp.float32),
          jax.ShapeDtypeStruct((n, _D), jnp.float32),
      ],
  )(acc2, accc2,
    _r2(c1['dec_g']), _r2(c1['dec_b']), c1['dec_W'], _r2(c1['dec_bias']),
    _r2(p['bnE0_g']), _r2(p['bnE0_b']),
    _r2(c2['enc_g']), _r2(c2['enc_b']), c2['enc_W'], _r2(c2['enc_bias']))

  acc3, accc3 = sc_scatter(table2, pk1)

  edge_score, edge_feat = pl.pallas_call(
      _stage_d_body,
      out_shape=[
          jax.ShapeDtypeStruct((nseg, 2), jnp.float32),
          jax.ShapeDtypeStruct((nseg, _D), jnp.float32),
      ],
  )(acc3, accc3,
    _r2(c2['dec_g']), _r2(c2['dec_b']), c2['dec_W'], _r2(c2['dec_bias']),
    h1,
    p['cls_W1'], _r2(p['cls_b1']), _r2(p['cls_lng']), _r2(p['cls_lnb']),
    p['cls_W2'], _r2(p['cls_b2']))

  return (edge_score, edge_feat, node_feat, norm)


# speedup vs baseline: 8.0986x; 1.0030x over previous
import functools

import jax
import jax.numpy as jnp
from jax import lax
from jax.experimental import pallas as pl
from jax.experimental.pallas import tpu as pltpu
from jax.experimental.pallas import tpu_sc as plsc

_L = 16
_CHUNK = 128
_D = 128
_CW = 16
_NC = 2
_NS = 16
_NW = _NC * _NS


def _make_sc_scatter(e_total, nseg):
  nchunks = e_total // _CHUNK
  ept = nchunks // _NW
  extra = nchunks - ept * _NW
  rows_per_tile = -(-nseg // _NS)
  rows_per_tile = -(-rows_per_tile // _CHUNK) * _CHUNK
  nseg_pad = rows_per_tile * _NS
  tail = nseg - (nseg // _CHUNK) * _CHUNK
  mesh = plsc.VectorSubcoreMesh(
      core_axis_name="c", subcore_axis_name="s",
      num_cores=_NC, num_subcores=_NS)

  @functools.partial(
      pl.kernel,
      out_type=(
          jax.ShapeDtypeStruct((_NC, nseg, _D), jnp.float32),
          jax.ShapeDtypeStruct((_NC, nseg, _CW), jnp.float32),
      ),
      mesh=mesh,
      compiler_params=pltpu.CompilerParams(use_tc_tiling_on_sc=False,
                                           needs_layout_passes=False),
      scratch_types=[
          pltpu.VMEM((2, 3, _CHUNK), jnp.int32),
          pltpu.VMEM((_CHUNK, _D), jnp.float32),
          pltpu.VMEM((_CHUNK, _D), jnp.float32),
          pltpu.VMEM((2, _CHUNK), jnp.int32),
          pltpu.VMEM((_CHUNK, _CW), jnp.float32),
          pltpu.VMEM_SHARED((nseg_pad, _D), jnp.float32),
          pltpu.VMEM_SHARED((nseg_pad, _CW), jnp.float32),
          pltpu.SemaphoreType.DMA,
          pltpu.SemaphoreType.DMA,
          pltpu.SemaphoreType.DMA,
          pltpu.SemaphoreType.DMA,
          pltpu.SemaphoreType.DMA,
          pltpu.SemaphoreType.DMA,
          pltpu.SemaphoreType.DMA,
          pltpu.SemaphoreType.DMA,
      ],
  )
  def k(table_hbm, pk_hbm, out_hbm, outc_hbm,
        pk_v, msg0, msg1, dst_c, cnt_v, acc, accc,
        isem0, isem1, gsem0, gsem1, ssem0, ssem1, csem0, csem1):
    cid = lax.axis_index("c")
    sid = lax.axis_index("s")
    wid = sid * _NC + cid
    msg = (msg0, msg1)
    isem = (isem0, isem1)
    gsem = (gsem0, gsem1)
    ssem = (ssem0, ssem1)
    csem = (csem0, csem1)
    cbase = wid * ept

    def issue_idx(c, j):
      pltpu.async_copy(pk_hbm.at[cbase + c], pk_v.at[j], isem[j])

    def drain_idx(j):
      pltpu.make_async_copy(pk_hbm.at[0], pk_v.at[j], isem[j]).wait()

    def issue_gather(j):
      pltpu.async_copy(table_hbm.at[pk_v.at[j, 0]], msg[j], gsem[j])

    def drain_gather(j):
      pltpu.make_async_copy(table_hbm.at[pl.ds(0, _CHUNK), :], msg[j],
                            gsem[j]).wait()

    def issue_scatter(j):
      pltpu.async_copy(msg[j], acc.at[dst_c.at[j]], ssem[j], add=True)
      pltpu.async_copy(cnt_v, accc.at[dst_c.at[j]], csem[j], add=True)

    def drain_scatter(j):
      pltpu.make_async_copy(out_hbm.at[0, pl.ds(0, _CHUNK), :], msg[j],
                            ssem[j]).wait()
      pltpu.make_async_copy(outc_hbm.at[0, pl.ds(0, _CHUNK), :], cnt_v,
                            csem[j]).wait()

    issue_idx(0, 0)
    issue_idx(1, 1)

    zero = jnp.zeros((_L,), jnp.float32)

    def zero_body(i, carry):
      for j in range(_D // _L):
        msg0[i, pl.ds(j * _L, _L)] = zero
      cnt_v[i, pl.ds(0, _L)] = zero
      return carry

    lax.fori_loop(0, _CHUNK, zero_body, 0)

    base_row = sid * rows_per_tile
    for off in range(0, rows_per_tile, _CHUNK):
      pltpu.sync_copy(msg0.at[pl.ds(0, _CHUNK), :],
                      acc.at[pl.ds(base_row + off, _CHUNK), :])
      pltpu.sync_copy(cnt_v, accc.at[pl.ds(base_row + off, _CHUNK), :])

    ones = jnp.ones((_L,), jnp.float32)

    def ones_body(i, carry):
      cnt_v[i, pl.ds(0, _L)] = ones
      return carry

    lax.fori_loop(0, _CHUNK, ones_body, 0)
    plsc.subcore_barrier()

    drain_idx(0)
    issue_gather(0)

    def do_chunk(c, b):
      drain_gather(b)

      for j in range(_CHUNK // _L):
        dst_c[b, pl.ds(j * _L, _L)] = pk_v[b, 1, pl.ds(j * _L, _L)]

      @plsc.parallel_loop(0, _CHUNK // _L, 1, unroll=2)
      def scale(g):
        nv16 = plsc.bitcast(pk_v[b, 2, pl.ds(g * _L, _L)], jnp.float32)
        for t in range(_L):
          s = nv16[t]
          row = g * _L + t
          for j in range(_D // _L):
            msg[b][row, pl.ds(j * _L, _L)] = s * msg[b][row, pl.ds(j * _L, _L)]
      issue_scatter(b)

      @pl.when(c >= 1)
      def _():
        drain_scatter(1 - b)

      @pl.when(c + 1 < ept)
      def _():
        drain_idx(1 - b)
        issue_gather(1 - b)

      @pl.when(c + 2 < ept)
      def _():
        issue_idx(c + 2, b)

    def outer(g, carry):
      do_chunk(g * 2, 0)
      do_chunk(g * 2 + 1, 1)
      return carry

    lax.fori_loop(0, ept // 2, outer, 0)
    if ept % 2:
      do_chunk(ept - 1, (ept - 1) % 2)
    if ept >= 1:
      drain_scatter((ept - 1) % 2)

    if extra:
      @pl.when(wid < extra)
      def _():
        xc = ept * _NW + wid
        pltpu.sync_copy(pk_hbm.at[xc], pk_v.at[0])
        pltpu.async_copy(table_hbm.at[pk_v.at[0, 0]], msg0, gsem0).wait()
        for j in range(_CHUNK // _L):
          dst_c[0, pl.ds(j * _L, _L)] = pk_v[0, 1, pl.ds(j * _L, _L)]

        @plsc.parallel_loop(0, _CHUNK // _L, 1, unroll=2)
        def xscale(g):
          nv16 = plsc.bitcast(pk_v[0, 2, pl.ds(g * _L, _L)], jnp.float32)
          for t in range(_L):
            s = nv16[t]
            row = g * _L + t
            for j in range(_D // _L):
              msg0[row, pl.ds(j * _L, _L)] = s * msg0[row, pl.ds(j * _L, _L)]
        pltpu.sync_copy(msg0, acc.at[dst_c.at[0]], add=True)
        pltpu.sync_copy(cnt_v, accc.at[dst_c.at[0]], add=True)

    plsc.subcore_barrier()

    for off in range(0, rows_per_tile, _CHUNK):
      start = base_row + off

      @pl.when(start + _CHUNK <= nseg)
      def _():
        pltpu.sync_copy(acc.at[pl.ds(start, _CHUNK), :],
                        msg0.at[pl.ds(0, _CHUNK), :])
        pltpu.sync_copy(msg0.at[pl.ds(0, _CHUNK), :],
                        out_hbm.at[cid, pl.ds(start, _CHUNK), :])
        pltpu.sync_copy(accc.at[pl.ds(start, _CHUNK), :], cnt_v)
        pltpu.sync_copy(cnt_v, outc_hbm.at[cid, pl.ds(start, _CHUNK), :])

      if tail:
        @pl.when((start < nseg) & (start + _CHUNK > nseg))
        def _():
          pltpu.sync_copy(acc.at[pl.ds(start, tail), :],
                          msg0.at[pl.ds(0, tail), :])
          pltpu.sync_copy(msg0.at[pl.ds(0, tail), :],
                          out_hbm.at[cid, pl.ds(start, tail), :])
          pltpu.sync_copy(accc.at[pl.ds(start, tail), :],
                          cnt_v.at[pl.ds(0, tail), :])
          pltpu.sync_copy(cnt_v.at[pl.ds(0, tail), :],
                          outc_hbm.at[cid, pl.ds(start, tail), :])

  return k


def _ln(x, g, b, eps=1e-5):
  m = jnp.mean(x, axis=-1, keepdims=True)
  v = jnp.mean((x - m) ** 2, axis=-1, keepdims=True)
  return g * (x - m) / jnp.sqrt(v + eps) + b


def _center_scale(x):
  x = x - jnp.mean(x, axis=0, keepdims=True)
  return x / jnp.sqrt(1e-5 + jnp.mean(jnp.sum(x * x, axis=-1)))


def _dense(x, g, b, w, bias):
  return jnp.maximum(
      jnp.dot(_ln(x, g, b), w, preferred_element_type=jnp.float32) + bias, 0.0)


def _stage_a_body(x_ref, g_ref, b_ref, w_ref, bias_ref, ei0_ref, ei1_ref,
                  nrm_ref, h_ref, pk1_ref, pk2_ref):
  h_ref[...] = _dense(x_ref[...], g_ref[...], b_ref[...], w_ref[...],
                      bias_ref[...])
  row0 = ei0_ref[...]
  ei1 = ei1_ref[...]
  row1 = ei1 - jnp.min(ei1)
  nrm_bits = jax.lax.bitcast_convert_type(nrm_ref[...], jnp.int32)
  pk1_ref[...] = jnp.stack([row0, row1, nrm_bits], axis=1)
  pk2_ref[...] = jnp.stack([row1, row0, nrm_bits], axis=1)


def _agg_decode(acc, accc, dec_g, dec_b, dec_w, dec_bias):
  s = acc[0] + acc[1]
  cnt = (accc[0] + accc[1])[:, 0:1]
  agg = s / jnp.maximum(cnt, 1.0)
  return _center_scale(_dense(agg, dec_g, dec_b, dec_w, dec_bias))


def _stage_mid_body(acc_ref, accc_ref, dec_g, dec_b, dec_w, dec_bias,
                    bn_g, bn_b, enc_g, enc_b, enc_w, enc_bias,
                    h_ref, table_ref):
  h = _agg_decode(acc_ref[...], accc_ref[...], dec_g[...], dec_b[...],
                  dec_w[...], dec_bias[...])
  h_ref[...] = h
  t = jnp.maximum(bn_g[...] * h / jnp.sqrt(1.0 + 1e-5) + bn_b[...], 0.0)
  table_ref[...] = _dense(t, enc_g[...], enc_b[...], enc_w[...], enc_bias[...])


def _stage_d_body(acc_ref, accc_ref, dec_g, dec_b, dec_w, dec_bias, h1_ref,
                  w1_ref, b1_ref, lng_ref, lnb_ref, w2_ref, b2_ref,
                  score_ref, h3_ref):
  h3 = _agg_decode(acc_ref[...], accc_ref[...], dec_g[...], dec_b[...],
                   dec_w[...], dec_bias[...])
  h3_ref[...] = h3
  xc = jnp.concatenate([h1_ref[...], h3], axis=1)
  hcl = jnp.maximum(
      jnp.dot(xc, w1_ref[...], preferred_element_type=jnp.float32)
      + b1_ref[...], 0.0)
  hcl = _ln(hcl, lng_ref[...], lnb_ref[...])
  score_ref[...] = (jnp.dot(hcl, w2_ref[...],
                            preferred_element_type=jnp.float32) + b2_ref[...])


def _r2(v):
  return v.reshape(1, -1)


def kernel(x, edge_index, norm, params):
  n, _ = x.shape
  e = edge_index.shape[1]
  nseg = 10000
  nch = e // _CHUNK
  p = params

  ei0_2d = edge_index[0].reshape(nch, _CHUNK)
  ei1_2d = edge_index[1].reshape(nch, _CHUNK)
  nrm_2d = norm.reshape(nch, _CHUNK)

  c0, c1, c2 = p['v2e0'], p['e2v0'], p['v2e1']

  h0, pk1, pk2 = pl.pallas_call(
      _stage_a_body,
      out_shape=[
          jax.ShapeDtypeStruct((n, _D), jnp.float32),
          jax.ShapeDtypeStruct((nch, 3, _CHUNK), jnp.int32),
          jax.ShapeDtypeStruct((nch, 3, _CHUNK), jnp.int32),
      ],
  )(x, _r2(c0['enc_g']), _r2(c0['enc_b']), c0['enc_W'], _r2(c0['enc_bias']),
    ei0_2d, ei1_2d, nrm_2d)

  sc_scatter = _make_sc_scatter(e, nseg)

  acc1, accc1 = sc_scatter(h0, pk1)

  h1, table1 = pl.pallas_call(
      _stage_mid_body,
      out_shape=[
          jax.ShapeDtypeStruct((nseg, _D), jnp.float32),
          jax.ShapeDtypeStruct((nseg, _D), jnp.float32),
      ],
  )(acc1, accc1,
    _r2(c0['dec_g']), _r2(c0['dec_b']), c0['dec_W'], _r2(c0['dec_bias']),
    _r2(p['bnV0_g']), _r2(p['bnV0_b']),
    _r2(c1['enc_g']), _r2(c1['enc_b']), c1['enc_W'], _r2(c1['enc_bias']))

  acc2, accc2 = sc_scatter(table1, pk2)

  node_feat, table2 = pl.pallas_call(
      _stage_mid_body,
      out_shape=[
          jax.ShapeDtypeStruct((n, _D), jnp.float32),
          jax.ShapeDtypeStruct((n, _D), jnp.float32),
      ],
  )(acc2, accc2,
    _r2(c1['dec_g']), _r2(c1['dec_b']), c1['dec_W'], _r2(c1['dec_bias']),
    _r2(p['bnE0_g']), _r2(p['bnE0_b']),
    _r2(c2['enc_g']), _r2(c2['enc_b']), c2['enc_W'], _r2(c2['enc_bias']))

  acc3, accc3 = sc_scatter(table2, pk1)

  edge_score, edge_feat = pl.pallas_call(
      _stage_d_body,
      out_shape=[
          jax.ShapeDtypeStruct((nseg, 2), jnp.float32),
          jax.ShapeDtypeStruct((nseg, _D), jnp.float32),
      ],
  )(acc3, accc3,
    _r2(c2['dec_g']), _r2(c2['dec_b']), c2['dec_W'], _r2(c2['dec_bias']),
    h1,
    p['cls_W1'], _r2(p['cls_b1']), _r2(p['cls_lng']), _r2(p['cls_lnb']),
    p['cls_W2'], _r2(p['cls_b2']))

  return (edge_score, edge_feat, node_feat, norm)


# speedup vs baseline: 9.7403x; 1.2027x over previous
import functools

import jax
import jax.numpy as jnp
from jax import lax
from jax.experimental import pallas as pl
from jax.experimental.pallas import tpu as pltpu
from jax.experimental.pallas import tpu_sc as plsc

_L = 16
_CHUNK = 128
_D = 128
_CW = 16
_NC = 2
_NS = 16
_NW = _NC * _NS


def _make_sc_scatter(e_total, nseg):
  nchunks = e_total // _CHUNK
  ept = nchunks // _NW
  extra = nchunks - ept * _NW
  rows_per_tile = -(-nseg // _NS)
  rows_per_tile = -(-rows_per_tile // _CHUNK) * _CHUNK
  nseg_pad = rows_per_tile * _NS
  tail = nseg - (nseg // _CHUNK) * _CHUNK
  mesh = plsc.VectorSubcoreMesh(
      core_axis_name="c", subcore_axis_name="s",
      num_cores=_NC, num_subcores=_NS)

  @functools.partial(
      pl.kernel,
      out_type=(
          jax.ShapeDtypeStruct((_NC, nseg, _D), jnp.float32),
          jax.ShapeDtypeStruct((_NC, nseg, _CW), jnp.float32),
      ),
      mesh=mesh,
      compiler_params=pltpu.CompilerParams(use_tc_tiling_on_sc=False,
                                           needs_layout_passes=False),
      scratch_types=[
          pltpu.VMEM((2, 3, _CHUNK), jnp.int32),
          pltpu.VMEM((_CHUNK, _D), jnp.float32),
          pltpu.VMEM((_CHUNK, _D), jnp.float32),
          pltpu.VMEM((2, _CHUNK), jnp.int32),
          pltpu.VMEM((_CHUNK, _CW), jnp.float32),
          pltpu.VMEM_SHARED((nseg_pad, _D), jnp.float32),
          pltpu.VMEM_SHARED((nseg_pad, _CW), jnp.float32),
          pltpu.SemaphoreType.DMA,
          pltpu.SemaphoreType.DMA,
          pltpu.SemaphoreType.DMA,
          pltpu.SemaphoreType.DMA,
          pltpu.SemaphoreType.DMA,
          pltpu.SemaphoreType.DMA,
          pltpu.SemaphoreType.DMA,
          pltpu.SemaphoreType.DMA,
      ],
  )
  def k(table_hbm, pk_hbm, out_hbm, outc_hbm,
        pk_v, msg0, msg1, dst_c, cnt_v, acc, accc,
        isem0, isem1, gsem0, gsem1, ssem0, ssem1, csem0, csem1):
    cid = lax.axis_index("c")
    sid = lax.axis_index("s")
    wid = sid * _NC + cid
    msg = (msg0, msg1)
    isem = (isem0, isem1)
    gsem = (gsem0, gsem1)
    ssem = (ssem0, ssem1)
    csem = (csem0, csem1)
    cbase = wid * ept

    def issue_idx(c, j):
      pltpu.async_copy(pk_hbm.at[cbase + c], pk_v.at[j], isem[j])

    def drain_idx(j):
      pltpu.make_async_copy(pk_hbm.at[0], pk_v.at[j], isem[j]).wait()

    def issue_gather(j):
      pltpu.async_copy(table_hbm.at[pk_v.at[j, 0]], msg[j], gsem[j])

    def drain_gather(j):
      pltpu.make_async_copy(table_hbm.at[pl.ds(0, _CHUNK), :], msg[j],
                            gsem[j]).wait()

    def issue_scatter(j):
      pltpu.async_copy(msg[j], acc.at[dst_c.at[j]], ssem[j], add=True)
      pltpu.async_copy(cnt_v, accc.at[dst_c.at[j]], csem[j], add=True)

    def drain_scatter(j):
      pltpu.make_async_copy(out_hbm.at[0, pl.ds(0, _CHUNK), :], msg[j],
                            ssem[j]).wait()
      pltpu.make_async_copy(outc_hbm.at[0, pl.ds(0, _CHUNK), :], cnt_v,
                            csem[j]).wait()

    issue_idx(0, 0)
    issue_idx(1, 1)

    zero = jnp.zeros((_L,), jnp.float32)

    def zero_body(i, carry):
      for j in range(_D // _L):
        msg0[i, pl.ds(j * _L, _L)] = zero
      cnt_v[i, pl.ds(0, _L)] = zero
      return carry

    lax.fori_loop(0, _CHUNK, zero_body, 0)

    base_row = sid * rows_per_tile
    for off in range(0, rows_per_tile, _CHUNK):
      pltpu.sync_copy(msg0.at[pl.ds(0, _CHUNK), :],
                      acc.at[pl.ds(base_row + off, _CHUNK), :])
      pltpu.sync_copy(cnt_v, accc.at[pl.ds(base_row + off, _CHUNK), :])

    ones = jnp.ones((_L,), jnp.float32)

    def ones_body(i, carry):
      cnt_v[i, pl.ds(0, _L)] = ones
      return carry

    lax.fori_loop(0, _CHUNK, ones_body, 0)
    plsc.subcore_barrier()

    drain_idx(0)
    issue_gather(0)

    def do_chunk(c, b):
      drain_gather(b)

      for j in range(_CHUNK // _L):
        dst_c[b, pl.ds(j * _L, _L)] = pk_v[b, 1, pl.ds(j * _L, _L)]

      @pl.when(c >= 1)
      def _():
        drain_scatter(1 - b)

      @pl.when(c + 1 < ept)
      def _():
        drain_idx(1 - b)
        issue_gather(1 - b)

      @plsc.parallel_loop(0, _CHUNK // _L, 1, unroll=2)
      def scale(g):
        nv16 = plsc.bitcast(pk_v[b, 2, pl.ds(g * _L, _L)], jnp.float32)
        for t in range(_L):
          s = nv16[t]
          row = g * _L + t
          for j in range(_D // _L):
            msg[b][row, pl.ds(j * _L, _L)] = s * msg[b][row, pl.ds(j * _L, _L)]
      issue_scatter(b)

      @pl.when(c + 2 < ept)
      def _():
        issue_idx(c + 2, b)

    def outer(g, carry):
      do_chunk(g * 2, 0)
      do_chunk(g * 2 + 1, 1)
      return carry

    lax.fori_loop(0, ept // 2, outer, 0)
    if ept % 2:
      do_chunk(ept - 1, (ept - 1) % 2)
    if ept >= 1:
      drain_scatter((ept - 1) % 2)

    if extra:
      @pl.when(wid < extra)
      def _():
        xc = ept * _NW + wid
        pltpu.sync_copy(pk_hbm.at[xc], pk_v.at[0])
        pltpu.async_copy(table_hbm.at[pk_v.at[0, 0]], msg0, gsem0).wait()
        for j in range(_CHUNK // _L):
          dst_c[0, pl.ds(j * _L, _L)] = pk_v[0, 1, pl.ds(j * _L, _L)]

        @plsc.parallel_loop(0, _CHUNK // _L, 1, unroll=2)
        def xscale(g):
          nv16 = plsc.bitcast(pk_v[0, 2, pl.ds(g * _L, _L)], jnp.float32)
          for t in range(_L):
            s = nv16[t]
            row = g * _L + t
            for j in range(_D // _L):
              msg0[row, pl.ds(j * _L, _L)] = s * msg0[row, pl.ds(j * _L, _L)]
        pltpu.sync_copy(msg0, acc.at[dst_c.at[0]], add=True)
        pltpu.sync_copy(cnt_v, accc.at[dst_c.at[0]], add=True)

    plsc.subcore_barrier()

    for off in range(0, rows_per_tile, _CHUNK):
      start = base_row + off

      @pl.when(start + _CHUNK <= nseg)
      def _():
        pltpu.sync_copy(acc.at[pl.ds(start, _CHUNK), :],
                        msg0.at[pl.ds(0, _CHUNK), :])
        pltpu.sync_copy(msg0.at[pl.ds(0, _CHUNK), :],
                        out_hbm.at[cid, pl.ds(start, _CHUNK), :])
        pltpu.sync_copy(accc.at[pl.ds(start, _CHUNK), :], cnt_v)
        pltpu.sync_copy(cnt_v, outc_hbm.at[cid, pl.ds(start, _CHUNK), :])

      if tail:
        @pl.when((start < nseg) & (start + _CHUNK > nseg))
        def _():
          pltpu.sync_copy(acc.at[pl.ds(start, tail), :],
                          msg0.at[pl.ds(0, tail), :])
          pltpu.sync_copy(msg0.at[pl.ds(0, tail), :],
                          out_hbm.at[cid, pl.ds(start, tail), :])
          pltpu.sync_copy(accc.at[pl.ds(start, tail), :],
                          cnt_v.at[pl.ds(0, tail), :])
          pltpu.sync_copy(cnt_v.at[pl.ds(0, tail), :],
                          outc_hbm.at[cid, pl.ds(start, tail), :])

  return k


def _ln(x, g, b, eps=1e-5):
  m = jnp.mean(x, axis=-1, keepdims=True)
  v = jnp.mean((x - m) ** 2, axis=-1, keepdims=True)
  return g * (x - m) / jnp.sqrt(v + eps) + b


def _center_scale(x):
  x = x - jnp.mean(x, axis=0, keepdims=True)
  return x / jnp.sqrt(1e-5 + jnp.mean(jnp.sum(x * x, axis=-1)))


def _dense(x, g, b, w, bias):
  return jnp.maximum(
      jnp.dot(_ln(x, g, b), w, preferred_element_type=jnp.float32) + bias, 0.0)


def _stage_a_body(x_ref, g_ref, b_ref, w_ref, bias_ref, ei0_ref, ei1_ref,
                  nrm_ref, h_ref, pk1_ref, pk2_ref):
  h_ref[...] = _dense(x_ref[...], g_ref[...], b_ref[...], w_ref[...],
                      bias_ref[...])
  row0 = ei0_ref[...]
  ei1 = ei1_ref[...]
  row1 = ei1 - jnp.min(ei1)
  nrm_bits = jax.lax.bitcast_convert_type(nrm_ref[...], jnp.int32)
  pk1_ref[...] = jnp.stack([row0, row1, nrm_bits], axis=1)
  pk2_ref[...] = jnp.stack([row1, row0, nrm_bits], axis=1)


def _agg_decode(acc, accc, dec_g, dec_b, dec_w, dec_bias):
  s = acc[0] + acc[1]
  cnt = (accc[0] + accc[1])[:, 0:1]
  agg = s / jnp.maximum(cnt, 1.0)
  return _center_scale(_dense(agg, dec_g, dec_b, dec_w, dec_bias))


def _stage_mid_body(acc_ref, accc_ref, dec_g, dec_b, dec_w, dec_bias,
                    bn_g, bn_b, enc_g, enc_b, enc_w, enc_bias,
                    h_ref, table_ref):
  h = _agg_decode(acc_ref[...], accc_ref[...], dec_g[...], dec_b[...],
                  dec_w[...], dec_bias[...])
  h_ref[...] = h
  t = jnp.maximum(bn_g[...] * h / jnp.sqrt(1.0 + 1e-5) + bn_b[...], 0.0)
  table_ref[...] = _dense(t, enc_g[...], enc_b[...], enc_w[...], enc_bias[...])


def _stage_d_body(acc_ref, accc_ref, dec_g, dec_b, dec_w, dec_bias, h1_ref,
                  w1_ref, b1_ref, lng_ref, lnb_ref, w2_ref, b2_ref,
                  score_ref, h3_ref):
  h3 = _agg_decode(acc_ref[...], accc_ref[...], dec_g[...], dec_b[...],
                   dec_w[...], dec_bias[...])
  h3_ref[...] = h3
  xc = jnp.concatenate([h1_ref[...], h3], axis=1)
  hcl = jnp.maximum(
      jnp.dot(xc, w1_ref[...], preferred_element_type=jnp.float32)
      + b1_ref[...], 0.0)
  hcl = _ln(hcl, lng_ref[...], lnb_ref[...])
  score_ref[...] = (jnp.dot(hcl, w2_ref[...],
                            preferred_element_type=jnp.float32) + b2_ref[...])


def _r2(v):
  return v.reshape(1, -1)


def kernel(x, edge_index, norm, params):
  n, _ = x.shape
  e = edge_index.shape[1]
  nseg = 10000
  nch = e // _CHUNK
  p = params

  ei0_2d = edge_index[0].reshape(nch, _CHUNK)
  ei1_2d = edge_index[1].reshape(nch, _CHUNK)
  nrm_2d = norm.reshape(nch, _CHUNK)

  c0, c1, c2 = p['v2e0'], p['e2v0'], p['v2e1']

  h0, pk1, pk2 = pl.pallas_call(
      _stage_a_body,
      out_shape=[
          jax.ShapeDtypeStruct((n, _D), jnp.float32),
          jax.ShapeDtypeStruct((nch, 3, _CHUNK), jnp.int32),
          jax.ShapeDtypeStruct((nch, 3, _CHUNK), jnp.int32),
      ],
  )(x, _r2(c0['enc_g']), _r2(c0['enc_b']), c0['enc_W'], _r2(c0['enc_bias']),
    ei0_2d, ei1_2d, nrm_2d)

  sc_scatter = _make_sc_scatter(e, nseg)

  acc1, accc1 = sc_scatter(h0, pk1)

  h1, table1 = pl.pallas_call(
      _stage_mid_body,
      out_shape=[
          jax.ShapeDtypeStruct((nseg, _D), jnp.float32),
          jax.ShapeDtypeStruct((nseg, _D), jnp.float32),
      ],
  )(acc1, accc1,
    _r2(c0['dec_g']), _r2(c0['dec_b']), c0['dec_W'], _r2(c0['dec_bias']),
    _r2(p['bnV0_g']), _r2(p['bnV0_b']),
    _r2(c1['enc_g']), _r2(c1['enc_b']), c1['enc_W'], _r2(c1['enc_bias']))

  acc2, accc2 = sc_scatter(table1, pk2)

  node_feat, table2 = pl.pallas_call(
      _stage_mid_body,
      out_shape=[
          jax.ShapeDtypeStruct((n, _D), jnp.float32),
          jax.ShapeDtypeStruct((n, _D), jnp.float32),
      ],
  )(acc2, accc2,
    _r2(c1['dec_g']), _r2(c1['dec_b']), c1['dec_W'], _r2(c1['dec_bias']),
    _r2(p['bnE0_g']), _r2(p['bnE0_b']),
    _r2(c2['enc_g']), _r2(c2['enc_b']), c2['enc_W'], _r2(c2['enc_bias']))

  acc3, accc3 = sc_scatter(table2, pk1)

  edge_score, edge_feat = pl.pallas_call(
      _stage_d_body,
      out_shape=[
          jax.ShapeDtypeStruct((nseg, 2), jnp.float32),
          jax.ShapeDtypeStruct((nseg, _D), jnp.float32),
      ],
  )(acc3, accc3,
    _r2(c2['dec_g']), _r2(c2['dec_b']), c2['dec_W'], _r2(c2['dec_bias']),
    h1,
    p['cls_W1'], _r2(p['cls_b1']), _r2(p['cls_lng']), _r2(p['cls_lnb']),
    p['cls_W2'], _r2(p['cls_b2']))

  return (edge_score, edge_feat, node_feat, norm)
